# Initial kernel scaffold; baseline (speedup 1.0000x reference)
import jax
import jax.numpy as jnp
from jax.experimental import pallas as pl


def kernel(x, table):
    raise NotImplementedError("write your pallas kernel here")


# speedup vs baseline: 1.3724x; 1.3724x over previous
import functools

import jax
import jax.numpy as jnp
from jax import lax
from jax.experimental import pallas as pl
from jax.experimental.pallas import tpu as pltpu
from jax.experimental.pallas import tpu_sc as plsc

VOCAB = 42000
D = 512
NC = 2
NS = 16
NW = NC * NS
B = 4096 * 50
BPW = B // NW
C = 80
NCHUNK = BPW // C

_mesh = plsc.VectorSubcoreMesh(core_axis_name="c", subcore_axis_name="s")


@functools.partial(
    pl.kernel,
    out_type=jax.ShapeDtypeStruct((B, D), jnp.float32),
    mesh=_mesh,
    scratch_types=[
        pltpu.VMEM((BPW,), jnp.int32),
        pltpu.VMEM((C, D), jnp.float32),
        pltpu.SemaphoreType.DMA,
    ],
)
def _gather_kernel(idx_hbm, table_hbm, out_hbm, idx_v, rows_v, sem):
    wid = lax.axis_index("s") * NC + lax.axis_index("c")
    base = wid * BPW
    pltpu.sync_copy(idx_hbm.at[pl.ds(base, BPW)], idx_v)

    def body(j, carry):
        off = j * C
        pltpu.async_copy(
            table_hbm.at[idx_v.at[pl.ds(off, C)]], rows_v, sem
        ).wait()
        pltpu.sync_copy(rows_v, out_hbm.at[pl.ds(base + off, C)])
        return carry

    lax.fori_loop(0, NCHUNK, body, 0)


def kernel(x, table):
    idx = x.reshape(B).astype(jnp.int32)
    out = _gather_kernel(idx, table)
    return out.reshape(x.shape[0], x.shape[1], D)


# speedup vs baseline: 1.4216x; 1.0358x over previous
import functools

import jax
import jax.numpy as jnp
from jax import lax
from jax.experimental import pallas as pl
from jax.experimental.pallas import tpu as pltpu
from jax.experimental.pallas import tpu_sc as plsc

VOCAB = 42000
D = 512
NC = 2
NS = 16
NW = NC * NS
B = 4096 * 50
BPW = B // NW
C = 80
NCHUNK = BPW // C
NBUF = 2
NITER = NCHUNK // NBUF

_mesh = plsc.VectorSubcoreMesh(core_axis_name="c", subcore_axis_name="s")


@functools.partial(
    pl.kernel,
    out_type=jax.ShapeDtypeStruct((B, D), jnp.float32),
    mesh=_mesh,
    scratch_types=[
        pltpu.VMEM((BPW,), jnp.int32),
        pltpu.VMEM((NBUF, C, D), jnp.float32),
        pltpu.SemaphoreType.DMA((NBUF,)),
        pltpu.SemaphoreType.DMA((NBUF,)),
    ],
)
def _gather_kernel(idx_hbm, table_hbm, out_hbm, idx_v, rows_v, gsem, osem):
    wid = lax.axis_index("s") * NC + lax.axis_index("c")
    base = wid * BPW
    pltpu.sync_copy(idx_hbm.at[pl.ds(base, BPW)], idx_v)

    def start_gather(j, b):
        pltpu.async_copy(
            table_hbm.at[idx_v.at[pl.ds(j * C, C)]], rows_v.at[b], gsem.at[b]
        )

    def wait_gather(b):
        pltpu.make_async_copy(
            table_hbm.at[pl.ds(0, C)], rows_v.at[b], gsem.at[b]
        ).wait()

    def start_out(j, b):
        pltpu.async_copy(
            rows_v.at[b], out_hbm.at[pl.ds(base + j * C, C)], osem.at[b]
        )

    def wait_out(b):
        pltpu.make_async_copy(
            rows_v.at[b], out_hbm.at[pl.ds(base, C)], osem.at[b]
        ).wait()

    for b in range(NBUF):
        start_gather(b, b)

    def body(g, carry):
        j0 = g * NBUF
        for b in range(NBUF):
            wait_gather(b)
            start_out(j0 + b, b)
        for b in range(NBUF):
            j2 = jnp.minimum(j0 + NBUF + b, NCHUNK - 1)
            wait_out(b)
            start_gather(j2, b)
        return carry

    lax.fori_loop(0, NITER, body, 0)

    for b in range(NBUF):
        wait_gather(b)


def kernel(x, table):
    idx = x.reshape(B).astype(jnp.int32)
    out = _gather_kernel(idx, table)
    return out.reshape(x.shape[0], x.shape[1], D)


# speedup vs baseline: 1.4332x; 1.0082x over previous
import functools

import jax
import jax.numpy as jnp
from jax import lax
from jax.experimental import pallas as pl
from jax.experimental.pallas import tpu as pltpu
from jax.experimental.pallas import tpu_sc as plsc

VOCAB = 42000
D = 512
NC = 2
NS = 16
NW = NC * NS
B = 4096 * 50
BPW = B // NW
C = 40
NCHUNK = BPW // C
NBUF = 4
SKEW = NBUF // 2
NGRP = NCHUNK // NBUF

_mesh = plsc.VectorSubcoreMesh(core_axis_name="c", subcore_axis_name="s")


@functools.partial(
    pl.kernel,
    out_type=jax.ShapeDtypeStruct((B, D), jnp.float32),
    mesh=_mesh,
    scratch_types=[
        pltpu.VMEM((BPW,), jnp.int32),
        pltpu.VMEM((NBUF, C, D), jnp.float32),
        pltpu.SemaphoreType.DMA((NBUF,)),
        pltpu.SemaphoreType.DMA((NBUF,)),
    ],
)
def _gather_kernel(idx_hbm, table_hbm, out_hbm, idx_v, rows_v, gsem, osem):
    wid = lax.axis_index("s") * NC + lax.axis_index("c")
    base = wid * BPW
    pltpu.sync_copy(idx_hbm.at[pl.ds(base, BPW)], idx_v)

    def start_gather(j, b):
        pltpu.async_copy(
            table_hbm.at[idx_v.at[pl.ds(j * C, C)]], rows_v.at[b], gsem.at[b]
        )

    def wait_gather(b):
        pltpu.make_async_copy(
            table_hbm.at[pl.ds(0, C)], rows_v.at[b], gsem.at[b]
        ).wait()

    def start_out(j, b):
        pltpu.async_copy(
            rows_v.at[b], out_hbm.at[pl.ds(base + j * C, C)], osem.at[b]
        )

    def wait_out(b):
        pltpu.make_async_copy(
            rows_v.at[b], out_hbm.at[pl.ds(base, C)], osem.at[b]
        ).wait()

    for b in range(SKEW):
        start_gather(b, b)

    for b in range(NBUF):
        wait_gather(b)
        start_out(b, b)
        b2 = (b + SKEW) % NBUF
        if b < SKEW:
            start_gather(b + SKEW, b2)
        else:
            wait_out(b2)
            start_gather(b + SKEW, b2)

    def body(g, carry):
        j0 = g * NBUF
        for b in range(NBUF):
            j = j0 + b
            wait_gather(b)
            start_out(j, b)
            b2 = (b + SKEW) % NBUF
            wait_out(b2)
            start_gather(jnp.minimum(j + SKEW, NCHUNK - 1), b2)
        return carry

    lax.fori_loop(1, NGRP, body, 0)

    for b in range(SKEW):
        wait_gather(b)
    for b in range(SKEW, NBUF):
        wait_out(b)


def kernel(x, table):
    idx = x.reshape(B).astype(jnp.int32)
    out = _gather_kernel(idx, table)
    return out.reshape(x.shape[0], x.shape[1], D)
